# Initial kernel scaffold; baseline (speedup 1.0000x reference)
import jax
import jax.numpy as jnp
from jax.experimental import pallas as pl


def kernel(contexts, table):
    raise NotImplementedError("write your pallas kernel here")


# speedup vs baseline: 1.0356x; 1.0356x over previous
import functools

import jax
import jax.numpy as jnp
from jax import lax
from jax.experimental import pallas as pl
from jax.experimental.pallas import tpu as pltpu
from jax.experimental.pallas import tpu_sc as plsc


@functools.lru_cache(maxsize=None)
def _make_sc_gather(N, V, D, C):
    info = plsc.get_sparse_core_info()
    NC, NS = info.num_cores, info.num_subcores
    NW = NC * NS
    assert N % NW == 0
    n_per_w = N // NW
    assert n_per_w % C == 0 and C % 8 == 0
    n_chunks = n_per_w // C
    assert n_chunks % 2 == 0 and n_chunks >= 2
    mesh = plsc.VectorSubcoreMesh(core_axis_name="c", subcore_axis_name="s")

    @functools.partial(
        pl.kernel,
        mesh=mesh,
        compiler_params=pltpu.CompilerParams(use_tc_tiling_on_sc=False),
        out_type=jax.ShapeDtypeStruct((N, D), jnp.float32),
        scratch_types=[
            pltpu.VMEM((n_per_w,), jnp.int32),
            pltpu.VMEM((C, D), jnp.float32),
            pltpu.VMEM((C, D), jnp.float32),
            pltpu.SemaphoreType.DMA,
            pltpu.SemaphoreType.DMA,
            pltpu.SemaphoreType.DMA,
            pltpu.SemaphoreType.DMA,
        ],
    )
    def gather_kernel(table_hbm, idx_hbm, out_hbm, idx_v, buf0, buf1,
                      gs0, gs1, ss0, ss1):
        wid = lax.axis_index("s") * NC + lax.axis_index("c")
        base = wid * n_per_w
        pltpu.sync_copy(idx_hbm.at[pl.ds(base, n_per_w)], idx_v)

        bufs = ((buf0, gs0, ss0), (buf1, gs1, ss1))

        def start_gather(i, buf, gs):
            pltpu.async_copy(table_hbm.at[idx_v.at[pl.ds(i * C, C)]], buf, gs)

        def wait_gather(buf, gs):
            pltpu.make_async_copy(
                table_hbm.at[idx_v.at[pl.ds(0, C)]], buf, gs).wait()

        def start_scatter(i, buf, ss):
            pltpu.async_copy(buf, out_hbm.at[pl.ds(base + i * C, C)], ss)

        def wait_scatter(buf, ss):
            pltpu.make_async_copy(buf, out_hbm.at[pl.ds(base, C)], ss).wait()

        start_gather(0, buf0, gs0)

        def pair_body(it, carry):
            for b in range(2):
                i = 2 * it + b
                cur_buf, cur_gs, cur_ss = bufs[b]
                nxt_buf, nxt_gs, nxt_ss = bufs[1 - b]

                @pl.when(i + 1 < n_chunks)
                def _():
                    @pl.when(i >= 1)
                    def _():
                        wait_scatter(nxt_buf, nxt_ss)

                    start_gather(i + 1, nxt_buf, nxt_gs)

                wait_gather(cur_buf, cur_gs)
                start_scatter(i, cur_buf, cur_ss)
            return carry

        lax.fori_loop(0, n_chunks // 2, pair_body, 0)
        wait_scatter(buf0, ss0)
        wait_scatter(buf1, ss1)

    return gather_kernel


def kernel(contexts, table):
    B, S = contexts.shape
    V, D = table.shape
    N = B * S
    idx = contexts.reshape(N).astype(jnp.int32)
    out = _make_sc_gather(N, V, D, 32)(table, idx)
    return out.reshape(B, S, D)
